# Initial kernel scaffold; baseline (speedup 1.0000x reference)
import jax
import jax.numpy as jnp
from jax.experimental import pallas as pl


def kernel(X, grid_part, grid_part_norm, part_abs_map, grid_abs_odd, bit_map):
    raise NotImplementedError("write your pallas kernel here")


# speedup vs baseline: 4.3998x; 4.3998x over previous
import jax
import jax.numpy as jnp
from jax import lax
from jax.experimental import pallas as pl

_PERM = (0, 2, 4, 6, 1, 3, 5, 7)
_R = 512


def _tc_body(xt_ref, gmat_ref, gaug_ref, vals_ref, idx_ref):
    xt = xt_ref[...]
    gmat = gmat_ref[...]
    gaug = gaug_ref[...]
    R = xt.shape[1]
    row_is7 = lax.broadcasted_iota(jnp.int32, (8, 1), 0) == 7

    def variant(shift, parity_bit):
        xv = xt + shift
        negi = (xv < 0.0).astype(jnp.int32)
        x_odd = jnp.sum(negi, axis=0, keepdims=True) & 1
        flip = 1.0 - 2.0 * x_odd.astype(jnp.float32)
        xpart = jnp.abs(xv)
        xpart = jnp.where(row_is7, xpart * flip, xpart)
        mask = 1.0 - 2.0 * negi.astype(jnp.float32)
        mask = jnp.where(row_is7, mask * flip, mask)
        mb = (negi[0:1] * 1 + negi[2:3] * 2 + negi[4:5] * 4 + negi[6:7] * 8
              + negi[1:2] * 16 + negi[3:4] * 32 + negi[5:6] * 64
              + (negi[7:8] ^ x_odd) * 128) ^ parity_bit
        scores = 2.0 * jax.lax.dot_general(
            gmat[:, :8].astype(jnp.bfloat16), xpart.astype(jnp.bfloat16),
            (((1,), (0,)), ((), ())),
            preferred_element_type=jnp.float32) - gmat[:, 8:9]
        smax = jnp.max(scores, axis=0, keepdims=True)
        cio = lax.broadcasted_iota(jnp.int32, scores.shape, 0)
        cand = jnp.where(scores == smax, cio, jnp.int32(2**30))
        qidx = jnp.min(cand, axis=0, keepdims=True)
        onehot = (cio == qidx).astype(jnp.float32)
        gath = jax.lax.dot_general(
            gaug, onehot, (((1,), (0,)), ((), ())),
            preferred_element_type=jnp.float32,
            precision=jax.lax.Precision.HIGHEST)
        roundout = gath[0:8]
        code_abs = gath[8:9].astype(jnp.int32)
        rbits = gath[9:10].astype(jnp.int32)
        vals = roundout * mask
        dv = xv - vals
        err2 = jnp.sum(dv * dv, axis=0, keepdims=True)
        idx = (code_abs << 8) | (rbits ^ mb)
        return vals, idx, err2

    pvals, pidx, perr = variant(0.25, 1)
    mvals, midx, merr = variant(-0.25, 0)
    which = perr < merr
    vals_ref[...] = jnp.where(which, pvals - 0.25, mvals + 0.25)
    idx_ref[...] = jnp.where(which, pidx, midx).reshape(1, 1, R)


def kernel(X, grid_part, grid_part_norm, part_abs_map, grid_abs_odd, bit_map):
    N = X.shape[0]
    C = grid_part.shape[0]
    CPAD = ((C + 127) // 128) * 128

    g_odd = grid_abs_odd[part_abs_map].astype(jnp.int32)
    perm = jnp.array(_PERM, dtype=jnp.int32)
    gneg = (grid_part[:, perm] < 0).astype(jnp.int32)
    rbits = (gneg[:, 0] * 1 + gneg[:, 1] * 2 + gneg[:, 2] * 4
             + gneg[:, 3] * 8 + gneg[:, 4] * 16 + gneg[:, 5] * 32
             + gneg[:, 6] * 64 + (gneg[:, 7] ^ g_odd) * 128)

    gmat = jnp.zeros((CPAD, 16), jnp.float32)
    gmat = gmat.at[:C, 0:8].set(grid_part)
    gmat = gmat.at[:C, 8].set(grid_part_norm)
    gmat = gmat.at[C:, 8].set(1e30)

    gaug = jnp.zeros((16, CPAD), jnp.float32)
    gaug = gaug.at[0:8, :C].set(grid_part.T)
    gaug = gaug.at[8, :C].set(part_abs_map.astype(jnp.float32))
    gaug = gaug.at[9, :C].set(rbits.astype(jnp.float32))

    XT = X.T
    nblk = N // _R
    vals_t, idx3 = pl.pallas_call(
        _tc_body,
        grid=(nblk,),
        in_specs=[
            pl.BlockSpec((8, _R), lambda i: (0, i)),
            pl.BlockSpec((CPAD, 16), lambda i: (0, 0)),
            pl.BlockSpec((16, CPAD), lambda i: (0, 0)),
        ],
        out_specs=[
            pl.BlockSpec((8, _R), lambda i: (0, i)),
            pl.BlockSpec((1, 1, _R), lambda i: (i, 0, 0)),
        ],
        out_shape=[
            jax.ShapeDtypeStruct((8, N), jnp.float32),
            jax.ShapeDtypeStruct((nblk, 1, _R), jnp.int32),
        ],
    )(XT, gmat, gaug)
    return vals_t.T, idx3.reshape(N)


# speedup vs baseline: 8.2292x; 1.8703x over previous
import jax
import jax.numpy as jnp
from jax import lax
from jax.experimental import pallas as pl

_PERM = (0, 2, 4, 6, 1, 3, 5, 7)
_R = 512


def _tc_body(xt_ref, gmat_ref, g2_ref, gaug_ref, vals_ref, idx_ref):
    xt = xt_ref[...]
    gmat = gmat_ref[...]
    gaug = gaug_ref[...]
    R = xt.shape[1]
    row_is7 = lax.broadcasted_iota(jnp.int32, (8, 1), 0) == 7

    def variant(shift, parity_bit):
        xv = xt + shift
        negi = (xv < 0.0).astype(jnp.int32)
        x_odd = jnp.sum(negi, axis=0, keepdims=True) & 1
        flip = 1.0 - 2.0 * x_odd.astype(jnp.float32)
        xpart = jnp.abs(xv)
        xpart = jnp.where(row_is7, xpart * flip, xpart)
        mask = 1.0 - 2.0 * negi.astype(jnp.float32)
        mask = jnp.where(row_is7, mask * flip, mask)
        mb = (negi[0:1] * 1 + negi[2:3] * 2 + negi[4:5] * 4 + negi[6:7] * 8
              + negi[1:2] * 16 + negi[3:4] * 32 + negi[5:6] * 64
              + (negi[7:8] ^ x_odd) * 128) ^ parity_bit
        scores = jax.lax.dot_general(
            g2_ref[...], xpart.astype(jnp.bfloat16),
            (((1,), (0,)), ((), ())),
            preferred_element_type=jnp.float32) - gmat[:, 8:9]
        smax = jnp.max(scores, axis=0, keepdims=True)
        cio = lax.broadcasted_iota(jnp.int32, scores.shape, 0)
        cand = jnp.where(scores == smax, cio, jnp.int32(2**30))
        qidx = jnp.min(cand, axis=0, keepdims=True)
        onehot = (cio == qidx).astype(jnp.float32)
        gath = jax.lax.dot_general(
            gaug, onehot, (((1,), (0,)), ((), ())),
            preferred_element_type=jnp.float32)
        roundout = gath[0:8]
        code_abs = gath[8:9].astype(jnp.int32)
        rbits = gath[9:10].astype(jnp.int32)
        vals = roundout * mask
        dv = xv - vals
        err2 = jnp.sum(dv * dv, axis=0, keepdims=True)
        idx = (code_abs << 8) | (rbits ^ mb)
        return vals, idx, err2

    pvals, pidx, perr = variant(0.25, 1)
    mvals, midx, merr = variant(-0.25, 0)
    which = perr < merr
    vals_ref[...] = jnp.where(which, pvals - 0.25, mvals + 0.25)
    idx_ref[...] = jnp.where(which, pidx, midx).reshape(1, 1, R)


def kernel(X, grid_part, grid_part_norm, part_abs_map, grid_abs_odd, bit_map):
    N = X.shape[0]
    C = grid_part.shape[0]
    CPAD = ((C + 127) // 128) * 128

    g_odd = grid_abs_odd[part_abs_map].astype(jnp.int32)
    perm = jnp.array(_PERM, dtype=jnp.int32)
    gneg = (grid_part[:, perm] < 0).astype(jnp.int32)
    rbits = (gneg[:, 0] * 1 + gneg[:, 1] * 2 + gneg[:, 2] * 4
             + gneg[:, 3] * 8 + gneg[:, 4] * 16 + gneg[:, 5] * 32
             + gneg[:, 6] * 64 + (gneg[:, 7] ^ g_odd) * 128)

    gmat = jnp.zeros((CPAD, 16), jnp.float32)
    gmat = gmat.at[:C, 0:8].set(grid_part)
    gmat = gmat.at[:C, 8].set(grid_part_norm)
    gmat = gmat.at[C:, 8].set(1e30)
    g2 = jnp.zeros((CPAD, 8), jnp.bfloat16)
    g2 = g2.at[:C].set((2.0 * grid_part).astype(jnp.bfloat16))

    gaug = jnp.zeros((16, CPAD), jnp.float32)
    gaug = gaug.at[0:8, :C].set(grid_part.T)
    gaug = gaug.at[8, :C].set(part_abs_map.astype(jnp.float32))
    gaug = gaug.at[9, :C].set(rbits.astype(jnp.float32))

    XT = X.T
    nblk = N // _R
    vals_t, idx3 = pl.pallas_call(
        _tc_body,
        grid=(nblk,),
        in_specs=[
            pl.BlockSpec((8, _R), lambda i: (0, i)),
            pl.BlockSpec((CPAD, 16), lambda i: (0, 0)),
            pl.BlockSpec((CPAD, 8), lambda i: (0, 0)),
            pl.BlockSpec((16, CPAD), lambda i: (0, 0)),
        ],
        out_specs=[
            pl.BlockSpec((8, _R), lambda i: (0, i)),
            pl.BlockSpec((1, 1, _R), lambda i: (i, 0, 0)),
        ],
        out_shape=[
            jax.ShapeDtypeStruct((8, N), jnp.float32),
            jax.ShapeDtypeStruct((nblk, 1, _R), jnp.int32),
        ],
    )(XT, gmat, g2, gaug)
    return vals_t.T, idx3.reshape(N)


# speedup vs baseline: 9.2794x; 1.1276x over previous
import functools

import jax
import jax.numpy as jnp
from jax import lax
from jax.experimental import pallas as pl
from jax.experimental.pallas import tpu as pltpu
from jax.experimental.pallas import tpu_sc as plsc

_PERM = (0, 2, 4, 6, 1, 3, 5, 7)
_BITPOS = (0, 4, 1, 5, 2, 6, 3, 7)
_R = 512
_N_TILES = 32
_L = 16


def _tc_body(xt_ref, gmat_ref, g2_ref, cp_ref, cm_ref):
    xt = xt_ref[...]
    gmat = gmat_ref[...]
    R = xt.shape[1]

    def variant(shift, parity_bit):
        xv = xt + shift
        negi = (xv < 0.0).astype(jnp.int32)
        x_odd = jnp.sum(negi, axis=0, keepdims=True) & 1
        flip = 1.0 - 2.0 * x_odd.astype(jnp.float32)
        xpart = jnp.abs(xv)
        row_is7 = lax.broadcasted_iota(jnp.int32, (8, 1), 0) == 7
        xpart = jnp.where(row_is7, xpart * flip, xpart)
        mb = (negi[0:1] * 1 + negi[2:3] * 2 + negi[4:5] * 4 + negi[6:7] * 8
              + negi[1:2] * 16 + negi[3:4] * 32 + negi[5:6] * 64
              + (negi[7:8] ^ x_odd) * 128) ^ parity_bit
        scores = jax.lax.dot_general(
            g2_ref[...], xpart.astype(jnp.bfloat16),
            (((1,), (0,)), ((), ())),
            preferred_element_type=jnp.float32) - gmat[:, 8:9]
        smax = jnp.max(scores, axis=0, keepdims=True)
        cio = lax.broadcasted_iota(jnp.int32, scores.shape, 0)
        cand = jnp.where(scores == smax, cio, jnp.int32(2**30))
        qidx = jnp.min(cand, axis=0, keepdims=True)
        return (qidx << 8) | mb

    cp_ref[...] = variant(0.25, 1).reshape(1, 1, R)
    cm_ref[...] = variant(-0.25, 0).reshape(1, 1, R)


def _sc_kernel(xt_hbm, cp_hbm, cm_hbm, gt_hbm, code_hbm,
               valst_hbm, idx_hbm,
               xt_v, cp_v, cm_v, gt_v, code_v, vout_v, iout_v):
    n = xt_hbm.shape[1]
    rows = n // _N_TILES
    wid = lax.axis_index("s") * 2 + lax.axis_index("c")
    base = wid * rows
    pltpu.sync_copy(gt_hbm, gt_v)
    pltpu.sync_copy(code_hbm, code_v)
    pltpu.sync_copy(xt_hbm.at[:, pl.ds(base, rows)], xt_v)
    pltpu.sync_copy(cp_hbm.at[pl.ds(base, rows)], cp_v)
    pltpu.sync_copy(cm_hbm.at[pl.ds(base, rows)], cm_v)

    def group(g, _):
        o = g * _L

        def variant(cbuf, parity_bit, shift):
            cv = cbuf[pl.ds(o, _L)]
            q = lax.shift_right_logical(cv, 8)
            mb = cv & 255
            mm = mb ^ parity_bit
            code = plsc.load_gather(code_v, [q]) ^ mb
            err = jnp.zeros((_L,), jnp.float32)
            vals = []
            for d in range(8):
                r = plsc.load_gather(gt_v, [q * 8 + d])
                bit = lax.shift_right_logical(mm, _BITPOS[d]) & 1
                sgn = 1.0 - 2.0 * bit.astype(jnp.float32)
                v = r * sgn
                xv = xt_v[d, pl.ds(o, _L)] + shift
                diff = xv - v
                err = err + diff * diff
                vals.append(v)
            return code, err, vals

        code_p, err_p, vp = variant(cp_v, 1, 0.25)
        code_m, err_m, vm = variant(cm_v, 0, -0.25)
        which = err_p < err_m
        for d in range(8):
            vout_v[d, pl.ds(o, _L)] = jnp.where(which, vp[d] - 0.25, vm[d] + 0.25)
        iout_v[pl.ds(o, _L)] = jnp.where(which, code_p, code_m)
        return 0

    lax.fori_loop(0, rows // _L, group, 0)
    pltpu.sync_copy(vout_v, valst_hbm.at[:, pl.ds(base, rows)])
    pltpu.sync_copy(iout_v, idx_hbm.at[pl.ds(base, rows)])


def kernel(X, grid_part, grid_part_norm, part_abs_map, grid_abs_odd, bit_map):
    N = X.shape[0]
    C = grid_part.shape[0]
    CPAD = ((C + 127) // 128) * 128

    g_odd = grid_abs_odd[part_abs_map].astype(jnp.int32)
    perm = jnp.array(_PERM, dtype=jnp.int32)
    gneg = (grid_part[:, perm] < 0).astype(jnp.int32)
    rbits = (gneg[:, 0] * 1 + gneg[:, 1] * 2 + gneg[:, 2] * 4
             + gneg[:, 3] * 8 + gneg[:, 4] * 16 + gneg[:, 5] * 32
             + gneg[:, 6] * 64 + (gneg[:, 7] ^ g_odd) * 128)

    gmat = jnp.zeros((CPAD, 16), jnp.float32)
    gmat = gmat.at[:C, 8].set(grid_part_norm)
    gmat = gmat.at[C:, 8].set(1e30)
    g2 = jnp.zeros((CPAD, 8), jnp.bfloat16)
    g2 = g2.at[:C].set((2.0 * grid_part).astype(jnp.bfloat16))

    CPAD8 = ((C + 15) // 16) * 16
    gt = jnp.zeros((CPAD8, 8), jnp.float32).at[:C].set(grid_part).reshape(-1)
    code = jnp.zeros((CPAD8,), jnp.int32).at[:C].set(
        (part_abs_map.astype(jnp.int32) << 8) | rbits)

    XT = X.T
    nblk = N // _R
    cp3, cm3 = pl.pallas_call(
        _tc_body,
        grid=(nblk,),
        in_specs=[
            pl.BlockSpec((8, _R), lambda i: (0, i)),
            pl.BlockSpec((CPAD, 16), lambda i: (0, 0)),
            pl.BlockSpec((CPAD, 8), lambda i: (0, 0)),
        ],
        out_specs=[
            pl.BlockSpec((1, 1, _R), lambda i: (i, 0, 0)),
            pl.BlockSpec((1, 1, _R), lambda i: (i, 0, 0)),
        ],
        out_shape=[
            jax.ShapeDtypeStruct((nblk, 1, _R), jnp.int32),
            jax.ShapeDtypeStruct((nblk, 1, _R), jnp.int32),
        ],
    )(XT, gmat, g2)
    cp = cp3.reshape(N)
    cm = cm3.reshape(N)

    rows = N // _N_TILES
    sc = functools.partial(
        pl.kernel,
        out_type=[
            jax.ShapeDtypeStruct((8, N), jnp.float32),
            jax.ShapeDtypeStruct((N,), jnp.int32),
        ],
        mesh=plsc.VectorSubcoreMesh(core_axis_name="c", subcore_axis_name="s"),
        compiler_params=pltpu.CompilerParams(needs_layout_passes=False),
        scratch_types=[
            pltpu.VMEM((8, rows), jnp.float32),
            pltpu.VMEM((rows,), jnp.int32),
            pltpu.VMEM((rows,), jnp.int32),
            pltpu.VMEM((gt.shape[0],), jnp.float32),
            pltpu.VMEM((code.shape[0],), jnp.int32),
            pltpu.VMEM((8, rows), jnp.float32),
            pltpu.VMEM((rows,), jnp.int32),
        ],
    )(_sc_kernel)
    vals_t, idx = sc(XT, cp, cm, gt, code)
    return vals_t.T, idx


# speedup vs baseline: 10.1660x; 1.0955x over previous
import functools

import jax
import jax.numpy as jnp
from jax import lax
from jax.experimental import pallas as pl
from jax.experimental.pallas import tpu as pltpu
from jax.experimental.pallas import tpu_sc as plsc

_PERM = (0, 2, 4, 6, 1, 3, 5, 7)
_BITPOS = (0, 4, 1, 5, 2, 6, 3, 7)
_R = 1024
_N_TILES = 32
_L = 16


def _tc_body(xt_ref, gmat_ref, g2_ref, cp_ref, cm_ref):
    xt = xt_ref[...]
    gmat = gmat_ref[...]
    R = xt.shape[1]
    row_is7 = lax.broadcasted_iota(jnp.int32, (8, 1), 0) == 7

    def prep(shift, parity_bit):
        xv = xt + shift
        negi = (xv < 0.0).astype(jnp.int32)
        x_odd = jnp.sum(negi, axis=0, keepdims=True) & 1
        flip = 1.0 - 2.0 * x_odd.astype(jnp.float32)
        xpart = jnp.abs(xv)
        xpart = jnp.where(row_is7, xpart * flip, xpart)
        mb = (negi[0:1] * 1 + negi[2:3] * 2 + negi[4:5] * 4 + negi[6:7] * 8
              + negi[1:2] * 16 + negi[3:4] * 32 + negi[5:6] * 64
              + (negi[7:8] ^ x_odd) * 128) ^ parity_bit
        scores = jax.lax.dot_general(
            g2_ref[...], xpart.astype(jnp.bfloat16),
            (((1,), (0,)), ((), ())),
            preferred_element_type=jnp.float32) - gmat[:, 8:9]
        return scores, mb

    def reduce(scores, mb):
        smax = jnp.max(scores, axis=0, keepdims=True)
        cio = lax.broadcasted_iota(jnp.int32, scores.shape, 0)
        cand = jnp.where(scores == smax, cio, jnp.int32(2**30))
        qidx = jnp.min(cand, axis=0, keepdims=True)
        return (qidx << 8) | mb

    sp, mbp = prep(0.25, 1)
    sm, mbm = prep(-0.25, 0)
    cp_ref[...] = reduce(sp, mbp).reshape(1, 1, R)
    cm_ref[...] = reduce(sm, mbm).reshape(1, 1, R)


def _sc_kernel(xt_hbm, cp_hbm, cm_hbm, gt_hbm, code_hbm,
               valst_hbm, idx_hbm,
               xt_v, cp_v, cm_v, gt_v, code_v, vout_v, iout_v):
    n = xt_hbm.shape[1]
    rows = n // _N_TILES
    wid = lax.axis_index("s") * 2 + lax.axis_index("c")
    base = wid * rows
    pltpu.sync_copy(gt_hbm, gt_v)
    pltpu.sync_copy(code_hbm, code_v)
    pltpu.sync_copy(xt_hbm.at[:, pl.ds(base, rows)], xt_v)
    pltpu.sync_copy(cp_hbm.at[pl.ds(base, rows)], cp_v)
    pltpu.sync_copy(cm_hbm.at[pl.ds(base, rows)], cm_v)

    def group(g, _):
        o = g * _L

        def variant(cbuf, parity_bit, shift):
            cv = cbuf[pl.ds(o, _L)]
            q = lax.shift_right_logical(cv, 8)
            mb = cv & 255
            mm = mb ^ parity_bit
            code = plsc.load_gather(code_v, [q]) ^ mb
            err = jnp.zeros((_L,), jnp.float32)
            vals = []
            for d in range(8):
                r = plsc.load_gather(gt_v, [q * 8 + d])
                bit = lax.shift_right_logical(mm, _BITPOS[d]) & 1
                sgn = 1.0 - 2.0 * bit.astype(jnp.float32)
                v = r * sgn
                xv = xt_v[d, pl.ds(o, _L)] + shift
                diff = xv - v
                err = err + diff * diff
                vals.append(v)
            return code, err, vals

        code_p, err_p, vp = variant(cp_v, 1, 0.25)
        code_m, err_m, vm = variant(cm_v, 0, -0.25)
        which = err_p < err_m
        for d in range(8):
            vout_v[d, pl.ds(o, _L)] = jnp.where(which, vp[d] - 0.25, vm[d] + 0.25)
        iout_v[pl.ds(o, _L)] = jnp.where(which, code_p, code_m)
        return 0

    lax.fori_loop(0, rows // _L, group, 0)
    pltpu.sync_copy(vout_v, valst_hbm.at[:, pl.ds(base, rows)])
    pltpu.sync_copy(iout_v, idx_hbm.at[pl.ds(base, rows)])


def kernel(X, grid_part, grid_part_norm, part_abs_map, grid_abs_odd, bit_map):
    N = X.shape[0]
    C = grid_part.shape[0]
    CPAD = ((C + 127) // 128) * 128

    g_odd = grid_abs_odd[part_abs_map].astype(jnp.int32)
    perm = jnp.array(_PERM, dtype=jnp.int32)
    gneg = (grid_part[:, perm] < 0).astype(jnp.int32)
    rbits = (gneg[:, 0] * 1 + gneg[:, 1] * 2 + gneg[:, 2] * 4
             + gneg[:, 3] * 8 + gneg[:, 4] * 16 + gneg[:, 5] * 32
             + gneg[:, 6] * 64 + (gneg[:, 7] ^ g_odd) * 128)

    gmat = jnp.zeros((CPAD, 16), jnp.float32)
    gmat = gmat.at[:C, 8].set(grid_part_norm)
    gmat = gmat.at[C:, 8].set(1e30)
    g2 = jnp.zeros((CPAD, 8), jnp.bfloat16)
    g2 = g2.at[:C].set((2.0 * grid_part).astype(jnp.bfloat16))

    CPAD8 = ((C + 15) // 16) * 16
    gt = jnp.zeros((CPAD8, 8), jnp.float32).at[:C].set(grid_part).reshape(-1)
    code = jnp.zeros((CPAD8,), jnp.int32).at[:C].set(
        (part_abs_map.astype(jnp.int32) << 8) | rbits)

    XT = X.T
    nblk = N // _R
    cp3, cm3 = pl.pallas_call(
        _tc_body,
        grid=(nblk,),
        in_specs=[
            pl.BlockSpec((8, _R), lambda i: (0, i)),
            pl.BlockSpec((CPAD, 16), lambda i: (0, 0)),
            pl.BlockSpec((CPAD, 8), lambda i: (0, 0)),
        ],
        out_specs=[
            pl.BlockSpec((1, 1, _R), lambda i: (i, 0, 0)),
            pl.BlockSpec((1, 1, _R), lambda i: (i, 0, 0)),
        ],
        out_shape=[
            jax.ShapeDtypeStruct((nblk, 1, _R), jnp.int32),
            jax.ShapeDtypeStruct((nblk, 1, _R), jnp.int32),
        ],
    )(XT, gmat, g2)
    cp = cp3.reshape(N)
    cm = cm3.reshape(N)

    rows = N // _N_TILES
    sc = functools.partial(
        pl.kernel,
        out_type=[
            jax.ShapeDtypeStruct((8, N), jnp.float32),
            jax.ShapeDtypeStruct((N,), jnp.int32),
        ],
        mesh=plsc.VectorSubcoreMesh(core_axis_name="c", subcore_axis_name="s"),
        compiler_params=pltpu.CompilerParams(needs_layout_passes=False),
        scratch_types=[
            pltpu.VMEM((8, rows), jnp.float32),
            pltpu.VMEM((rows,), jnp.int32),
            pltpu.VMEM((rows,), jnp.int32),
            pltpu.VMEM((gt.shape[0],), jnp.float32),
            pltpu.VMEM((code.shape[0],), jnp.int32),
            pltpu.VMEM((8, rows), jnp.float32),
            pltpu.VMEM((rows,), jnp.int32),
        ],
    )(_sc_kernel)
    vals_t, idx = sc(XT, cp, cm, gt, code)
    return vals_t.T, idx


# speedup vs baseline: 10.2499x; 1.0083x over previous
import functools

import jax
import jax.numpy as jnp
from jax import lax
from jax.experimental import pallas as pl
from jax.experimental.pallas import tpu as pltpu
from jax.experimental.pallas import tpu_sc as plsc

_PERM = (0, 2, 4, 6, 1, 3, 5, 7)
_BITPOS = (0, 4, 1, 5, 2, 6, 3, 7)
_R = 1024
_N_TILES = 32
_L = 16


def _tc_body(x_ref, g2_ref, cp_ref, cm_ref):
    xt = x_ref[...].T
    R = xt.shape[1]
    row_is7 = lax.broadcasted_iota(jnp.int32, (8, 1), 0) == 7
    ones = jnp.ones((1, R), jnp.bfloat16)

    def prep(shift, parity_bit):
        xv = xt + shift
        negi = (xv < 0.0).astype(jnp.int32)
        x_odd = jnp.sum(negi, axis=0, keepdims=True) & 1
        flip = 1.0 - 2.0 * x_odd.astype(jnp.float32)
        xpart = jnp.abs(xv)
        xpart = jnp.where(row_is7, xpart * flip, xpart)
        mb = (negi[0:1] * 1 + negi[2:3] * 2 + negi[4:5] * 4 + negi[6:7] * 8
              + negi[1:2] * 16 + negi[3:4] * 32 + negi[5:6] * 64
              + (negi[7:8] ^ x_odd) * 128) ^ parity_bit
        xaug = jnp.concatenate([xpart.astype(jnp.bfloat16), ones], axis=0)
        scores = jax.lax.dot_general(
            g2_ref[:, :9], xaug, (((1,), (0,)), ((), ())),
            preferred_element_type=jnp.float32)
        return scores, mb

    sp, mbp = prep(0.25, 1)
    sm, mbm = prep(-0.25, 0)
    sub8 = lax.broadcasted_iota(jnp.int32, (8, R), 0).astype(jnp.float32)

    def reduce(scores, mb):
        nch = scores.shape[0] // 8
        m = scores[0:8]
        bk = jnp.zeros((8, R), jnp.float32)
        for k in range(1, nch):
            v = scores[8 * k:8 * (k + 1)]
            upd = v > m
            m = jnp.maximum(m, v)
            bk = jnp.where(upd, jnp.float32(k), bk)
        c = bk * 8.0 + sub8
        for sh in (4, 2, 1):
            mr = jnp.roll(m, -sh, axis=0)
            cr = jnp.roll(c, -sh, axis=0)
            take = (mr > m) | ((mr == m) & (cr < c))
            m = jnp.where(take, mr, m)
            c = jnp.where(take, cr, c)
        qidx = c[0:1].astype(jnp.int32)
        return (qidx << 8) | mb

    cp_ref[...] = reduce(sp, mbp).reshape(1, 1, R)
    cm_ref[...] = reduce(sm, mbm).reshape(1, 1, R)


def _sc_kernel(x_hbm, cp_hbm, cm_hbm, gt_hbm, code_hbm,
               vals_hbm, idx_hbm,
               x_v, cp_v, cm_v, gt_v, code_v, vout_v, iout_v):
    n = cp_hbm.shape[0]
    rows = n // _N_TILES
    wid = lax.axis_index("s") * 2 + lax.axis_index("c")
    base = wid * rows
    pltpu.sync_copy(gt_hbm, gt_v)
    pltpu.sync_copy(code_hbm, code_v)
    pltpu.sync_copy(x_hbm.at[pl.ds(base * 8, rows * 8)], x_v)
    pltpu.sync_copy(cp_hbm.at[pl.ds(base, rows)], cp_v)
    pltpu.sync_copy(cm_hbm.at[pl.ds(base, rows)], cm_v)
    lane = lax.iota(jnp.int32, _L)

    def group(g, _):
        o = g * _L
        row8 = (o + lane) * 8

        def variant(cbuf, parity_bit, shift):
            cv = cbuf[pl.ds(o, _L)]
            q = lax.shift_right_logical(cv, 8)
            mb = cv & 255
            mm = mb ^ parity_bit
            code = plsc.load_gather(code_v, [q]) ^ mb
            err = jnp.zeros((_L,), jnp.float32)
            vals = []
            for d in range(8):
                r = plsc.load_gather(gt_v, [q * 8 + d])
                bit = lax.shift_right_logical(mm, _BITPOS[d]) & 1
                sgn = 1.0 - 2.0 * bit.astype(jnp.float32)
                v = r * sgn
                xv = plsc.load_gather(x_v, [row8 + d]) + shift
                diff = xv - v
                err = err + diff * diff
                vals.append(v)
            return code, err, vals

        code_p, err_p, vp = variant(cp_v, 1, 0.25)
        code_m, err_m, vm = variant(cm_v, 0, -0.25)
        which = err_p < err_m
        for d in range(8):
            out = jnp.where(which, vp[d] - 0.25, vm[d] + 0.25)
            plsc.store_scatter(vout_v, [row8 + d], out)
        iout_v[pl.ds(o, _L)] = jnp.where(which, code_p, code_m)
        return 0

    lax.fori_loop(0, rows // _L, group, 0)
    pltpu.sync_copy(vout_v, vals_hbm.at[pl.ds(base * 8, rows * 8)])
    pltpu.sync_copy(iout_v, idx_hbm.at[pl.ds(base, rows)])


def kernel(X, grid_part, grid_part_norm, part_abs_map, grid_abs_odd, bit_map):
    N = X.shape[0]
    C = grid_part.shape[0]
    CPAD = ((C + 127) // 128) * 128

    g_odd = grid_abs_odd[part_abs_map].astype(jnp.int32)
    perm = jnp.array(_PERM, dtype=jnp.int32)
    gneg = (grid_part[:, perm] < 0).astype(jnp.int32)
    rbits = (gneg[:, 0] * 1 + gneg[:, 1] * 2 + gneg[:, 2] * 4
             + gneg[:, 3] * 8 + gneg[:, 4] * 16 + gneg[:, 5] * 32
             + gneg[:, 6] * 64 + (gneg[:, 7] ^ g_odd) * 128)

    g2 = jnp.zeros((CPAD, 16), jnp.bfloat16)
    g2 = g2.at[:C, 0:8].set((2.0 * grid_part).astype(jnp.bfloat16))
    g2 = g2.at[:C, 8].set((-grid_part_norm).astype(jnp.bfloat16))
    g2 = g2.at[C:, 8].set(jnp.bfloat16(-1e30))

    CPAD8 = ((C + 15) // 16) * 16
    gt = jnp.zeros((CPAD8, 8), jnp.float32).at[:C].set(grid_part).reshape(-1)
    code = jnp.zeros((CPAD8,), jnp.int32).at[:C].set(
        (part_abs_map.astype(jnp.int32) << 8) | rbits)

    nblk = N // _R
    cp3, cm3 = pl.pallas_call(
        _tc_body,
        grid=(nblk,),
        in_specs=[
            pl.BlockSpec((_R, 8), lambda i: (i, 0)),
            pl.BlockSpec((CPAD, 16), lambda i: (0, 0)),
        ],
        out_specs=[
            pl.BlockSpec((1, 1, _R), lambda i: (i, 0, 0)),
            pl.BlockSpec((1, 1, _R), lambda i: (i, 0, 0)),
        ],
        out_shape=[
            jax.ShapeDtypeStruct((nblk, 1, _R), jnp.int32),
            jax.ShapeDtypeStruct((nblk, 1, _R), jnp.int32),
        ],
    )(X, g2)
    cp = cp3.reshape(N)
    cm = cm3.reshape(N)

    rows = N // _N_TILES
    sc = functools.partial(
        pl.kernel,
        out_type=[
            jax.ShapeDtypeStruct((N * 8,), jnp.float32),
            jax.ShapeDtypeStruct((N,), jnp.int32),
        ],
        mesh=plsc.VectorSubcoreMesh(core_axis_name="c", subcore_axis_name="s"),
        compiler_params=pltpu.CompilerParams(needs_layout_passes=False),
        scratch_types=[
            pltpu.VMEM((rows * 8,), jnp.float32),
            pltpu.VMEM((rows,), jnp.int32),
            pltpu.VMEM((rows,), jnp.int32),
            pltpu.VMEM((gt.shape[0],), jnp.float32),
            pltpu.VMEM((code.shape[0],), jnp.int32),
            pltpu.VMEM((rows * 8,), jnp.float32),
            pltpu.VMEM((rows,), jnp.int32),
        ],
    )(_sc_kernel)
    vals_flat, idx = sc(X.reshape(N * 8), cp, cm, gt, code)
    return vals_flat.reshape(N, 8), idx


# speedup vs baseline: 10.4371x; 1.0183x over previous
import functools

import jax
import jax.numpy as jnp
from jax import lax
from jax.experimental import pallas as pl
from jax.experimental.pallas import tpu as pltpu
from jax.experimental.pallas import tpu_sc as plsc

_PERM = (0, 2, 4, 6, 1, 3, 5, 7)
_BITPOS = (0, 4, 1, 5, 2, 6, 3, 7)
_R = 1024
_N_TILES = 32
_L = 16


def _tc_body(x_ref, g2_ref, cp_ref, cm_ref):
    xt = x_ref[...].T
    R = xt.shape[1]
    row_is7 = lax.broadcasted_iota(jnp.int32, (8, 1), 0) == 7
    ones = jnp.ones((1, R), jnp.bfloat16)

    def prep(shift, parity_bit):
        xv = xt + shift
        negi = (xv < 0.0).astype(jnp.int32)
        x_odd = jnp.sum(negi, axis=0, keepdims=True) & 1
        flip = 1.0 - 2.0 * x_odd.astype(jnp.float32)
        xpart = jnp.abs(xv)
        xpart = jnp.where(row_is7, xpart * flip, xpart)
        mb = (negi[0:1] * 1 + negi[2:3] * 2 + negi[4:5] * 4 + negi[6:7] * 8
              + negi[1:2] * 16 + negi[3:4] * 32 + negi[5:6] * 64
              + (negi[7:8] ^ x_odd) * 128) ^ parity_bit
        xaug = jnp.concatenate([xpart.astype(jnp.bfloat16), ones], axis=0)
        scores = jax.lax.dot_general(
            g2_ref[:, :9], xaug, (((1,), (0,)), ((), ())),
            preferred_element_type=jnp.float32)
        return scores, mb

    sp, mbp = prep(0.25, 1)
    sm, mbm = prep(-0.25, 0)
    sub8 = lax.broadcasted_iota(jnp.int32, (8, R), 0).astype(jnp.float32)

    def reduce(scores, mb):
        nch = scores.shape[0] // 8
        m = scores[0:8]
        bk = jnp.zeros((8, R), jnp.float32)
        for k in range(1, nch):
            v = scores[8 * k:8 * (k + 1)]
            upd = v > m
            m = jnp.maximum(m, v)
            bk = jnp.where(upd, jnp.float32(k), bk)
        c = bk * 8.0 + sub8
        for sh in (4, 2, 1):
            mr = jnp.roll(m, -sh, axis=0)
            cr = jnp.roll(c, -sh, axis=0)
            take = (mr > m) | ((mr == m) & (cr < c))
            m = jnp.where(take, mr, m)
            c = jnp.where(take, cr, c)
        qidx = c[0:1].astype(jnp.int32)
        return (qidx << 8) | mb

    cp_ref[...] = reduce(sp, mbp).reshape(1, 1, R)
    cm_ref[...] = reduce(sm, mbm).reshape(1, 1, R)


def _sc_kernel(x_hbm, cp_hbm, cm_hbm, gt_hbm, code_hbm,
               vals_hbm, idx_hbm,
               x_v, cp_v, cm_v, gt_v, code_v, vout_v, iout_v, sem):
    n = cp_hbm.shape[0]
    rows = n // _N_TILES
    wid = lax.axis_index("s") * 2 + lax.axis_index("c")
    base = wid * rows
    c1 = pltpu.async_copy(gt_hbm, gt_v, sem)
    c2 = pltpu.async_copy(code_hbm, code_v, sem)
    c3 = pltpu.async_copy(x_hbm.at[pl.ds(base * 8, rows * 8)], x_v, sem)
    c4 = pltpu.async_copy(cp_hbm.at[pl.ds(base, rows)], cp_v, sem)
    c5 = pltpu.async_copy(cm_hbm.at[pl.ds(base, rows)], cm_v, sem)
    c1.wait()
    c2.wait()
    c3.wait()
    c4.wait()
    c5.wait()
    lane = lax.iota(jnp.int32, _L)

    def group(g, _):
        o = g * _L
        row8 = (o + lane) * 8

        def variant(cbuf, parity_bit, shift):
            cv = cbuf[pl.ds(o, _L)]
            q = lax.shift_right_logical(cv, 8)
            mb = cv & 255
            mm = mb ^ parity_bit
            code = plsc.load_gather(code_v, [q]) ^ mb
            err = jnp.zeros((_L,), jnp.float32)
            vals = []
            for d in range(8):
                r = plsc.load_gather(gt_v, [q * 8 + d])
                bit = lax.shift_right_logical(mm, _BITPOS[d]) & 1
                sgn = 1.0 - 2.0 * bit.astype(jnp.float32)
                v = r * sgn
                xv = plsc.load_gather(x_v, [row8 + d]) + shift
                diff = xv - v
                err = err + diff * diff
                vals.append(v)
            return code, err, vals

        code_p, err_p, vp = variant(cp_v, 1, 0.25)
        code_m, err_m, vm = variant(cm_v, 0, -0.25)
        which = err_p < err_m
        for d in range(8):
            out = jnp.where(which, vp[d] - 0.25, vm[d] + 0.25)
            plsc.store_scatter(vout_v, [row8 + d], out)
        iout_v[pl.ds(o, _L)] = jnp.where(which, code_p, code_m)
        return 0

    lax.fori_loop(0, rows // _L, group, 0)
    o1 = pltpu.async_copy(vout_v, vals_hbm.at[pl.ds(base * 8, rows * 8)], sem)
    o2 = pltpu.async_copy(iout_v, idx_hbm.at[pl.ds(base, rows)], sem)
    o1.wait()
    o2.wait()


def kernel(X, grid_part, grid_part_norm, part_abs_map, grid_abs_odd, bit_map):
    N = X.shape[0]
    C = grid_part.shape[0]
    CPAD = ((C + 127) // 128) * 128

    g_odd = grid_abs_odd[part_abs_map].astype(jnp.int32)
    perm = jnp.array(_PERM, dtype=jnp.int32)
    gneg = (grid_part[:, perm] < 0).astype(jnp.int32)
    rbits = (gneg[:, 0] * 1 + gneg[:, 1] * 2 + gneg[:, 2] * 4
             + gneg[:, 3] * 8 + gneg[:, 4] * 16 + gneg[:, 5] * 32
             + gneg[:, 6] * 64 + (gneg[:, 7] ^ g_odd) * 128)

    g2 = jnp.zeros((CPAD, 16), jnp.bfloat16)
    g2 = g2.at[:C, 0:8].set((2.0 * grid_part).astype(jnp.bfloat16))
    g2 = g2.at[:C, 8].set((-grid_part_norm).astype(jnp.bfloat16))
    g2 = g2.at[C:, 8].set(jnp.bfloat16(-1e30))

    CPAD8 = ((C + 15) // 16) * 16
    gt = jnp.zeros((CPAD8, 8), jnp.float32).at[:C].set(grid_part).reshape(-1)
    code = jnp.zeros((CPAD8,), jnp.int32).at[:C].set(
        (part_abs_map.astype(jnp.int32) << 8) | rbits)

    nblk = N // _R
    cp3, cm3 = pl.pallas_call(
        _tc_body,
        grid=(nblk,),
        in_specs=[
            pl.BlockSpec((_R, 8), lambda i: (i, 0)),
            pl.BlockSpec((CPAD, 16), lambda i: (0, 0)),
        ],
        out_specs=[
            pl.BlockSpec((1, 1, _R), lambda i: (i, 0, 0)),
            pl.BlockSpec((1, 1, _R), lambda i: (i, 0, 0)),
        ],
        out_shape=[
            jax.ShapeDtypeStruct((nblk, 1, _R), jnp.int32),
            jax.ShapeDtypeStruct((nblk, 1, _R), jnp.int32),
        ],
    )(X, g2)
    cp = cp3.reshape(N)
    cm = cm3.reshape(N)

    rows = N // _N_TILES
    sc = functools.partial(
        pl.kernel,
        out_type=[
            jax.ShapeDtypeStruct((N * 8,), jnp.float32),
            jax.ShapeDtypeStruct((N,), jnp.int32),
        ],
        mesh=plsc.VectorSubcoreMesh(core_axis_name="c", subcore_axis_name="s"),
        compiler_params=pltpu.CompilerParams(needs_layout_passes=False),
        scratch_types=[
            pltpu.VMEM((rows * 8,), jnp.float32),
            pltpu.VMEM((rows,), jnp.int32),
            pltpu.VMEM((rows,), jnp.int32),
            pltpu.VMEM((gt.shape[0],), jnp.float32),
            pltpu.VMEM((code.shape[0],), jnp.int32),
            pltpu.VMEM((rows * 8,), jnp.float32),
            pltpu.VMEM((rows,), jnp.int32),
            pltpu.SemaphoreType.DMA,
        ],
    )(_sc_kernel)
    vals_flat, idx = sc(X.reshape(N * 8), cp, cm, gt, code)
    return vals_flat.reshape(N, 8), idx


# speedup vs baseline: 10.7756x; 1.0324x over previous
import functools

import jax
import jax.numpy as jnp
from jax import lax
from jax.experimental import pallas as pl
from jax.experimental.pallas import tpu as pltpu
from jax.experimental.pallas import tpu_sc as plsc

_PERM = (0, 2, 4, 6, 1, 3, 5, 7)
_BITPOS = (0, 4, 1, 5, 2, 6, 3, 7)
_R = 2048
_N_TILES = 32
_L = 16


def _tc_body(x_ref, g2_ref, cp_ref, cm_ref):
    xt = x_ref[...].T
    R = xt.shape[1]
    row_is7 = lax.broadcasted_iota(jnp.int32, (8, 1), 0) == 7
    ones = jnp.ones((1, R), jnp.bfloat16)

    def prep(shift, parity_bit):
        xv = xt + shift
        negi = (xv < 0.0).astype(jnp.int32)
        x_odd = jnp.sum(negi, axis=0, keepdims=True) & 1
        flip = 1.0 - 2.0 * x_odd.astype(jnp.float32)
        xpart = jnp.abs(xv)
        xpart = jnp.where(row_is7, xpart * flip, xpart)
        mb = (negi[0:1] * 1 + negi[2:3] * 2 + negi[4:5] * 4 + negi[6:7] * 8
              + negi[1:2] * 16 + negi[3:4] * 32 + negi[5:6] * 64
              + (negi[7:8] ^ x_odd) * 128) ^ parity_bit
        xaug = jnp.concatenate([xpart.astype(jnp.bfloat16), ones], axis=0)
        scores = jax.lax.dot_general(
            g2_ref[:, :9], xaug, (((1,), (0,)), ((), ())),
            preferred_element_type=jnp.float32)
        return scores, mb

    sp, mbp = prep(0.25, 1)
    sm, mbm = prep(-0.25, 0)
    sub8 = lax.broadcasted_iota(jnp.int32, (8, R), 0).astype(jnp.float32)

    def reduce(scores, mb):
        nch = scores.shape[0] // 8
        m = scores[0:8]
        bk = jnp.zeros((8, R), jnp.float32)
        for k in range(1, nch):
            v = scores[8 * k:8 * (k + 1)]
            upd = v > m
            m = jnp.maximum(m, v)
            bk = jnp.where(upd, jnp.float32(k), bk)
        c = bk * 8.0 + sub8
        for sh in (4, 2, 1):
            mr = jnp.roll(m, -sh, axis=0)
            cr = jnp.roll(c, -sh, axis=0)
            take = (mr > m) | ((mr == m) & (cr < c))
            m = jnp.where(take, mr, m)
            c = jnp.where(take, cr, c)
        qidx = c[0:1].astype(jnp.int32)
        return (qidx << 8) | mb

    cp_ref[...] = reduce(sp, mbp).reshape(1, 1, R)
    cm_ref[...] = reduce(sm, mbm).reshape(1, 1, R)


def _sc_kernel(x_hbm, cp_hbm, cm_hbm, gt_hbm, code_hbm,
               vals_hbm, idx_hbm,
               x_v, cp_v, cm_v, gt_v, code_v, vout_v, iout_v, sem):
    n = cp_hbm.shape[0]
    rows = n // _N_TILES
    wid = lax.axis_index("s") * 2 + lax.axis_index("c")
    base = wid * rows
    c1 = pltpu.async_copy(gt_hbm, gt_v, sem)
    c2 = pltpu.async_copy(code_hbm, code_v, sem)
    c3 = pltpu.async_copy(x_hbm.at[pl.ds(base * 8, rows * 8)], x_v, sem)
    c4 = pltpu.async_copy(cp_hbm.at[pl.ds(base, rows)], cp_v, sem)
    c5 = pltpu.async_copy(cm_hbm.at[pl.ds(base, rows)], cm_v, sem)
    c1.wait()
    c2.wait()
    c3.wait()
    c4.wait()
    c5.wait()
    lane = lax.iota(jnp.int32, _L)

    def group(g, _):
        o = g * _L
        row8 = (o + lane) * 8

        def variant(cbuf, parity_bit, shift):
            cv = cbuf[pl.ds(o, _L)]
            q = lax.shift_right_logical(cv, 8)
            mb = cv & 255
            mm = mb ^ parity_bit
            code = plsc.load_gather(code_v, [q]) ^ mb
            err = jnp.zeros((_L,), jnp.float32)
            vals = []
            for d in range(8):
                r = plsc.load_gather(gt_v, [q * 8 + d])
                bit = lax.shift_right_logical(mm, _BITPOS[d]) & 1
                sgn = 1.0 - 2.0 * bit.astype(jnp.float32)
                v = r * sgn
                xv = plsc.load_gather(x_v, [row8 + d]) + shift
                diff = xv - v
                err = err + diff * diff
                vals.append(v)
            return code, err, vals

        code_p, err_p, vp = variant(cp_v, 1, 0.25)
        code_m, err_m, vm = variant(cm_v, 0, -0.25)
        which = err_p < err_m
        for d in range(8):
            out = jnp.where(which, vp[d] - 0.25, vm[d] + 0.25)
            plsc.store_scatter(vout_v, [row8 + d], out)
        iout_v[pl.ds(o, _L)] = jnp.where(which, code_p, code_m)
        return 0

    lax.fori_loop(0, rows // _L, group, 0)
    o1 = pltpu.async_copy(vout_v, vals_hbm.at[pl.ds(base * 8, rows * 8)], sem)
    o2 = pltpu.async_copy(iout_v, idx_hbm.at[pl.ds(base, rows)], sem)
    o1.wait()
    o2.wait()


def kernel(X, grid_part, grid_part_norm, part_abs_map, grid_abs_odd, bit_map):
    N = X.shape[0]
    C = grid_part.shape[0]
    CPAD = ((C + 127) // 128) * 128

    g_odd = grid_abs_odd[part_abs_map].astype(jnp.int32)
    perm = jnp.array(_PERM, dtype=jnp.int32)
    gneg = (grid_part[:, perm] < 0).astype(jnp.int32)
    rbits = (gneg[:, 0] * 1 + gneg[:, 1] * 2 + gneg[:, 2] * 4
             + gneg[:, 3] * 8 + gneg[:, 4] * 16 + gneg[:, 5] * 32
             + gneg[:, 6] * 64 + (gneg[:, 7] ^ g_odd) * 128)

    g2 = jnp.zeros((CPAD, 16), jnp.bfloat16)
    g2 = g2.at[:C, 0:8].set((2.0 * grid_part).astype(jnp.bfloat16))
    g2 = g2.at[:C, 8].set((-grid_part_norm).astype(jnp.bfloat16))
    g2 = g2.at[C:, 8].set(jnp.bfloat16(-1e30))

    CPAD8 = ((C + 15) // 16) * 16
    gt = jnp.zeros((CPAD8, 8), jnp.float32).at[:C].set(grid_part).reshape(-1)
    code = jnp.zeros((CPAD8,), jnp.int32).at[:C].set(
        (part_abs_map.astype(jnp.int32) << 8) | rbits)

    nblk = N // _R
    cp3, cm3 = pl.pallas_call(
        _tc_body,
        grid=(nblk,),
        in_specs=[
            pl.BlockSpec((_R, 8), lambda i: (i, 0)),
            pl.BlockSpec((CPAD, 16), lambda i: (0, 0)),
        ],
        out_specs=[
            pl.BlockSpec((1, 1, _R), lambda i: (i, 0, 0)),
            pl.BlockSpec((1, 1, _R), lambda i: (i, 0, 0)),
        ],
        out_shape=[
            jax.ShapeDtypeStruct((nblk, 1, _R), jnp.int32),
            jax.ShapeDtypeStruct((nblk, 1, _R), jnp.int32),
        ],
    )(X, g2)
    cp = cp3.reshape(N)
    cm = cm3.reshape(N)

    rows = N // _N_TILES
    sc = functools.partial(
        pl.kernel,
        out_type=[
            jax.ShapeDtypeStruct((N * 8,), jnp.float32),
            jax.ShapeDtypeStruct((N,), jnp.int32),
        ],
        mesh=plsc.VectorSubcoreMesh(core_axis_name="c", subcore_axis_name="s"),
        compiler_params=pltpu.CompilerParams(needs_layout_passes=False),
        scratch_types=[
            pltpu.VMEM((rows * 8,), jnp.float32),
            pltpu.VMEM((rows,), jnp.int32),
            pltpu.VMEM((rows,), jnp.int32),
            pltpu.VMEM((gt.shape[0],), jnp.float32),
            pltpu.VMEM((code.shape[0],), jnp.int32),
            pltpu.VMEM((rows * 8,), jnp.float32),
            pltpu.VMEM((rows,), jnp.int32),
            pltpu.SemaphoreType.DMA,
        ],
    )(_sc_kernel)
    vals_flat, idx = sc(X.reshape(N * 8), cp, cm, gt, code)
    return vals_flat.reshape(N, 8), idx
